# Initial kernel scaffold; baseline (speedup 1.0000x reference)
import jax
import jax.numpy as jnp
from jax.experimental import pallas as pl


def kernel(x, edge_index, edge_attr, batch, W0, b0, We0, W1, b1, We1, W2, b2, We2):
    raise NotImplementedError("write your pallas kernel here")


# speedup vs baseline: 1.9904x; 1.9904x over previous
import functools

import jax
import jax.numpy as jnp
from jax import lax
from jax.experimental import pallas as pl
from jax.experimental.pallas import tpu as pltpu
from jax.experimental.pallas import tpu_sc as plsc

N = 10000
E = 320000
G = 128
NPAD = 10240

NC = 2
NS = 16
NP = NPAD // NS
EPT = E // NS
C = 80
NCHUNK = EPT // C

_mesh = plsc.VectorSubcoreMesh(core_axis_name="c", subcore_axis_name="s")


def _edge_pass_body(H, src_hbm, dst_hbm, e_a, e_b, h_a, h_b, zr_hbm,
                    out_a, out_b, idx_s, idx_d, erows, hrows, aggr_sh, sem):
    cid = lax.axis_index("c")
    sid = lax.axis_index("s")

    pltpu.sync_copy(zr_hbm, aggr_sh.at[pl.ds(sid * NP, NP)])
    plsc.subcore_barrier()

    def run(e_hbm, h_hbm, out_hbm):
        def chunk(k, carry):
            base = sid * EPT + k * C
            pltpu.sync_copy(src_hbm.at[pl.ds(base, C)], idx_s)
            pltpu.sync_copy(dst_hbm.at[pl.ds(base, C)], idx_d)
            pltpu.sync_copy(e_hbm.at[pl.ds(base, C)], erows)
            pltpu.async_copy(h_hbm.at[idx_s], hrows, sem).wait()

            def row(r, c2):
                for j in range(H // 16):
                    sl = pl.ds(j * 16, 16)
                    erows[r, sl] = jnp.maximum(erows[r, sl] + hrows[r, sl],
                                               0.0)
                return c2
            lax.fori_loop(0, C, row, 0)
            pltpu.sync_copy(erows, aggr_sh.at[idx_d], add=True)
            return carry
        lax.fori_loop(0, NCHUNK, chunk, 0)
        plsc.subcore_barrier()
        pltpu.sync_copy(aggr_sh.at[pl.ds(sid * NP, NP)],
                        out_hbm.at[pl.ds(sid * NP, NP)])

    pl.when(cid == 0)(lambda: run(e_a, h_a, out_a))
    pl.when(cid == 1)(lambda: run(e_b, h_b, out_b))


_edge_pass_128 = pl.kernel(
    functools.partial(_edge_pass_body, 128),
    out_type=(jax.ShapeDtypeStruct((NPAD, 128), jnp.float32),
              jax.ShapeDtypeStruct((NPAD, 128), jnp.float32)),
    mesh=_mesh,
    scratch_types=[
        pltpu.VMEM((C,), jnp.int32),
        pltpu.VMEM((C,), jnp.int32),
        pltpu.VMEM((C, 128), jnp.float32),
        pltpu.VMEM((C, 128), jnp.float32),
        pltpu.VMEM_SHARED((NPAD, 128), jnp.float32),
        pltpu.SemaphoreType.DMA,
    ],
)


EPT0 = E // (NC * NS)
NCHUNK0 = EPT0 // C


def _edge_pass0_body(src_hbm, dst_hbm, e_hbm, h_hbm, zr_hbm,
                     out_p0, out_p1, idx_s, idx_d, erows, hrows,
                     aggr_sh, sem):
    cid = lax.axis_index("c")
    sid = lax.axis_index("s")
    wid = cid * NS + sid

    pltpu.sync_copy(zr_hbm, aggr_sh.at[pl.ds(sid * NP, NP)])
    plsc.subcore_barrier()

    def chunk(k, carry):
        base = wid * EPT0 + k * C
        pltpu.sync_copy(src_hbm.at[pl.ds(base, C)], idx_s)
        pltpu.sync_copy(dst_hbm.at[pl.ds(base, C)], idx_d)
        pltpu.sync_copy(e_hbm.at[pl.ds(base, C)], erows)
        pltpu.async_copy(h_hbm.at[idx_s], hrows, sem).wait()

        def row(r, c2):
            for j in range(128 // 16):
                sl = pl.ds(j * 16, 16)
                erows[r, sl] = jnp.maximum(erows[r, sl] + hrows[r, sl], 0.0)
            return c2
        lax.fori_loop(0, C, row, 0)
        pltpu.sync_copy(erows, aggr_sh.at[idx_d], add=True)
        return carry
    lax.fori_loop(0, NCHUNK0, chunk, 0)
    plsc.subcore_barrier()

    def out(out_hbm):
        pltpu.sync_copy(aggr_sh.at[pl.ds(sid * NP, NP)],
                        out_hbm.at[pl.ds(sid * NP, NP)])
    pl.when(cid == 0)(lambda: out(out_p0))
    pl.when(cid == 1)(lambda: out(out_p1))


_edge_pass_l0 = pl.kernel(
    _edge_pass0_body,
    out_type=(jax.ShapeDtypeStruct((NPAD, 128), jnp.float32),
              jax.ShapeDtypeStruct((NPAD, 128), jnp.float32)),
    mesh=_mesh,
    scratch_types=[
        pltpu.VMEM((C,), jnp.int32),
        pltpu.VMEM((C,), jnp.int32),
        pltpu.VMEM((C, 128), jnp.float32),
        pltpu.VMEM((C, 128), jnp.float32),
        pltpu.VMEM_SHARED((NPAD, 128), jnp.float32),
        pltpu.SemaphoreType.DMA,
    ],
)


CP = 128
RPT = NPAD // NS
GPT = G // NS


def _pool_body(s_a, s_b, batch_hbm, mask_hbm, zr_hbm, zrc_hbm,
               pool_out_a, pool_out_b, cnt_out,
               idxb, rows, ones_rows, pool_sh, cnt_sh, sem):
    cid = lax.axis_index("c")
    sid = lax.axis_index("s")

    pltpu.sync_copy(zr_hbm, pool_sh.at[pl.ds(sid * GPT, GPT)])
    pl.when(cid == 0)(lambda: pltpu.sync_copy(
        zrc_hbm, cnt_sh.at[pl.ds(sid * GPT, GPT)]))
    plsc.subcore_barrier()

    def run(s_hbm):
        def chunk(k, carry):
            base = sid * RPT + k * CP
            pltpu.sync_copy(batch_hbm.at[pl.ds(base, CP)], idxb)
            pltpu.sync_copy(s_hbm.at[pl.ds(base, CP)], rows)
            pltpu.sync_copy(rows, pool_sh.at[idxb], add=True)
            return carry
        lax.fori_loop(0, RPT // CP, chunk, 0)

    def run_counts():
        def chunk(k, carry):
            base = sid * RPT + k * CP
            pltpu.sync_copy(batch_hbm.at[pl.ds(base, CP)], idxb)
            pltpu.sync_copy(mask_hbm.at[pl.ds(base, CP)], ones_rows)
            pltpu.sync_copy(ones_rows, cnt_sh.at[idxb], add=True)
            return carry
        lax.fori_loop(0, RPT // CP, chunk, 0)

    pl.when(cid == 0)(lambda: run(s_a))
    pl.when(cid == 1)(lambda: run(s_b))
    pl.when(cid == 0)(run_counts)
    plsc.subcore_barrier()

    def out(pool_out):
        pltpu.sync_copy(pool_sh.at[pl.ds(sid * GPT, GPT)],
                        pool_out.at[pl.ds(sid * GPT, GPT)])
    pl.when(cid == 0)(lambda: out(pool_out_a))
    pl.when(cid == 1)(lambda: out(pool_out_b))
    pl.when(cid == 0)(lambda: pltpu.sync_copy(
        cnt_sh.at[pl.ds(sid * GPT, GPT)], cnt_out.at[pl.ds(sid * GPT, GPT)]))


_pool = pl.kernel(
    _pool_body,
    out_type=(jax.ShapeDtypeStruct((G, 128), jnp.float32),
              jax.ShapeDtypeStruct((G, 128), jnp.float32),
              jax.ShapeDtypeStruct((G, 128), jnp.float32)),
    mesh=_mesh,
    scratch_types=[
        pltpu.VMEM((CP,), jnp.int32),
        pltpu.VMEM((CP, 128), jnp.float32),
        pltpu.VMEM((CP, 128), jnp.float32),
        pltpu.VMEM_SHARED((G, 128), jnp.float32),
        pltpu.VMEM_SHARED((G, 128), jnp.float32),
        pltpu.SemaphoreType.DMA,
    ],
)


EB = 3200


def _edge_mm_body(attr_ref, we0_ref, we1_ref, we2_ref,
                  e0, e1a, e1b, e2a, e2b):
    attr = attr_ref[...]
    dn = (((1,), (1,)), ((), ()))
    m0 = lax.dot_general(attr, we0_ref[...], dn,
                         preferred_element_type=jnp.float32)
    m1 = lax.dot_general(attr, we1_ref[...], dn,
                         preferred_element_type=jnp.float32)
    m2 = lax.dot_general(attr, we2_ref[...], dn,
                         preferred_element_type=jnp.float32)
    e0[...] = m0
    e1a[...] = m1[:, :128]
    e1b[...] = m1[:, 128:]
    e2a[...] = m2[:, :128]
    e2b[...] = m2[:, 128:]


def _edge_mm(attr, We0, We1, We2):
    grid = (E // EB,)
    return pl.pallas_call(
        _edge_mm_body,
        grid=grid,
        in_specs=[
            pl.BlockSpec((EB, 16), lambda i: (i, 0)),
            pl.BlockSpec((128, 16), lambda i: (0, 0)),
            pl.BlockSpec((256, 16), lambda i: (0, 0)),
            pl.BlockSpec((256, 16), lambda i: (0, 0)),
        ],
        out_specs=[
            pl.BlockSpec((EB, 128), lambda i: (i, 0)),
            pl.BlockSpec((EB, 128), lambda i: (i, 0)),
            pl.BlockSpec((EB, 128), lambda i: (i, 0)),
            pl.BlockSpec((EB, 128), lambda i: (i, 0)),
            pl.BlockSpec((EB, 128), lambda i: (i, 0)),
        ],
        out_shape=[
            jax.ShapeDtypeStruct((E, 128), jnp.float32),
            jax.ShapeDtypeStruct((E, 128), jnp.float32),
            jax.ShapeDtypeStruct((E, 128), jnp.float32),
            jax.ShapeDtypeStruct((E, 128), jnp.float32),
            jax.ShapeDtypeStruct((E, 128), jnp.float32),
        ],
    )(attr, We0, We1, We2)


NB = 2000


def _node_body(first, *refs):
    if first:
        h_a, h_b, g_a, g_b, wa, wb, b2, hn_a, hn_b, sn_a, sn_b = refs
    else:
        (h_a, h_b, g_a, g_b, wa, wb, b2, sp_a, sp_b,
         hn_a, hn_b, sn_a, sn_b) = refs
    dn = (((1,), (1,)), ((), ()))
    t = lax.dot_general(h_a[...] + g_a[...], wa[...], dn,
                        preferred_element_type=jnp.float32)
    t += lax.dot_general(h_b[...] + g_b[...], wb[...], dn,
                         preferred_element_type=jnp.float32)
    r = jnp.maximum(t + b2[...], 0.0)
    ra, rb = r[:, :128], r[:, 128:]
    hn_a[...] = ra
    hn_b[...] = rb
    if first:
        sn_a[...] = ra
        sn_b[...] = rb
    else:
        sn_a[...] = sp_a[...] + ra
        sn_b[...] = sp_b[...] + rb


def _node_update(h_a, h_b, g_a, g_b, Wa, Wb, b2, sp_a=None, sp_b=None,
                 first=False):
    H = h_a.shape[1]
    grid = (N // NB,)
    blk = lambda w: pl.BlockSpec((NB, w), lambda i: (i, 0))
    full = lambda a: pl.BlockSpec(a.shape, lambda i: (0,) * a.ndim)
    args = [h_a, h_b, g_a, g_b, Wa, Wb, b2]
    specs = [blk(H), blk(H), blk(H), blk(H), full(Wa), full(Wb), full(b2)]
    if not first:
        args += [sp_a, sp_b]
        specs += [blk(128), blk(128)]
    return pl.pallas_call(
        functools.partial(_node_body, first),
        grid=grid,
        in_specs=specs,
        out_specs=[blk(128), blk(128), blk(128), blk(128)],
        out_shape=[jax.ShapeDtypeStruct((N, 128), jnp.float32)] * 4,
    )(*args)


def _node0_body(x_ref, p0_ref, p1_ref, wa, wb, b2, hn_a, hn_b, sn_a, sn_b):
    h = x_ref[...] + p0_ref[...] + p1_ref[...]
    dn = (((1,), (1,)), ((), ()))
    t = lax.dot_general(h[:, :64], wa[...], dn,
                        preferred_element_type=jnp.float32)
    t += lax.dot_general(h[:, 64:], wb[...], dn,
                         preferred_element_type=jnp.float32)
    r = jnp.maximum(t + b2[...], 0.0)
    hn_a[...] = r[:, :128]
    hn_b[...] = r[:, 128:]
    sn_a[...] = r[:, :128]
    sn_b[...] = r[:, 128:]


def _node_update0(x, p0, p1, Wa, Wb, b2):
    grid = (N // NB,)
    blk = pl.BlockSpec((NB, 128), lambda i: (i, 0))
    full = lambda a: pl.BlockSpec(a.shape, lambda i: (0,) * a.ndim)
    return pl.pallas_call(
        _node0_body,
        grid=grid,
        in_specs=[blk, blk, blk, full(Wa), full(Wb), full(b2)],
        out_specs=[blk, blk, blk, blk],
        out_shape=[jax.ShapeDtypeStruct((N, 128), jnp.float32)] * 4,
    )(x, p0, p1, Wa, Wb, b2)


def _scale_body(pa_ref, pb_ref, cnt_ref, out_ref):
    cnt = cnt_ref[:, 0:1]
    scale = jnp.where(cnt > 0.0, lax.rsqrt(jnp.maximum(cnt, 1.0)), 0.0)
    out_ref[:, :128] = pa_ref[...] * scale
    out_ref[:, 128:] = pb_ref[...] * scale


def _scale(pool_a, pool_b, cnt):
    return pl.pallas_call(
        _scale_body,
        out_shape=jax.ShapeDtypeStruct((G, 256), jnp.float32),
    )(pool_a, pool_b, cnt)


def kernel(x, edge_index, edge_attr, batch, W0, b0, We0, W1, b1, We1,
           W2, b2, We2):
    src = edge_index[0]
    dst = edge_index[1]

    zr128 = jnp.zeros((NP, 128), jnp.float32)
    zr_g = jnp.zeros((GPT, 128), jnp.float32)
    zr_gc = jnp.zeros((GPT, 128), jnp.float32)
    batch_pad = jnp.concatenate(
        [batch, jnp.full((NPAD - N,), G - 1, jnp.int32)])
    mask_pad = jnp.concatenate(
        [jnp.ones((N, 128), jnp.float32), jnp.zeros((NPAD - N, 128),
                                                    jnp.float32)])

    e0, e1a, e1b, e2a, e2b = _edge_mm(edge_attr, We0, We1, We2)

    p0, p1 = _edge_pass_l0(src, dst, e0, x, zr128)
    h1a, h1b, s1a, s1b = _node_update0(
        x, p0, p1, W0[:, :64], W0[:, 64:], b0[None, :])

    g1a, g1b = _edge_pass_128(src, dst, e1a, e1b, h1a, h1b, zr128)
    h2a, h2b, s2a, s2b = _node_update(
        h1a, h1b, g1a, g1b, W1[:, :128], W1[:, 128:], b1[None, :],
        s1a, s1b, first=False)

    g2a, g2b = _edge_pass_128(src, dst, e2a, e2b, h2a, h2b, zr128)
    _, _, s3a, s3b = _node_update(
        h2a, h2b, g2a, g2b, W2[:, :128], W2[:, 128:], b2[None, :],
        s2a, s2b, first=False)

    s3a_pad = jnp.concatenate([s3a, jnp.zeros((NPAD - N, 128), jnp.float32)])
    s3b_pad = jnp.concatenate([s3b, jnp.zeros((NPAD - N, 128), jnp.float32)])

    pool_a, pool_b, cnt = _pool(s3a_pad, s3b_pad, batch_pad, mask_pad,
                                zr_g, zr_gc)
    return _scale(pool_a, pool_b, cnt)


# speedup vs baseline: 3.7177x; 1.8679x over previous
import functools

import jax
import jax.numpy as jnp
from jax import lax
from jax.experimental import pallas as pl
from jax.experimental.pallas import tpu as pltpu
from jax.experimental.pallas import tpu_sc as plsc

N = 10000
E = 320000
G = 128
NPAD = 10240

NC = 2
NS = 16
NP = NPAD // NS
EPT = E // NS
C = 80
NCHUNK = EPT // C

_mesh = plsc.VectorSubcoreMesh(core_axis_name="c", subcore_axis_name="s")


def _edge_chunks(H, CH, NCH, base, src_hbm, dst_hbm, e_hbm, h_hbm,
                 idx_s, idx_d, erows, hrows, sems, aggr_sh):
    def issue_idx(k, b):
        pltpu.async_copy(src_hbm.at[pl.ds(base + k * CH, CH)],
                         idx_s.at[b], sems.at[b])
        pltpu.async_copy(dst_hbm.at[pl.ds(base + k * CH, CH)],
                         idx_d.at[b], sems.at[b])
        pltpu.async_copy(e_hbm.at[pl.ds(base + k * CH, CH)],
                         erows.at[b], sems.at[2 + b])

    def wait_idx(k, b):
        pltpu.make_async_copy(src_hbm.at[pl.ds(base + k * CH, CH)],
                              idx_s.at[b], sems.at[b]).wait()
        pltpu.make_async_copy(dst_hbm.at[pl.ds(base + k * CH, CH)],
                              idx_d.at[b], sems.at[b]).wait()

    def issue_gather(b):
        pltpu.async_copy(h_hbm.at[idx_s.at[b]], hrows.at[b], sems.at[4 + b])

    issue_idx(0, 0)
    issue_idx(1, 1)
    wait_idx(0, 0)
    issue_gather(0)

    def pair(kk, carry):
        for b in range(2):
            k = kk * 2 + b
            b1 = 1 - b

            @pl.when(k + 1 < NCH)
            def _():
                wait_idx(k + 1, b1)
                issue_gather(b1)

            pltpu.make_async_copy(e_hbm.at[pl.ds(base + k * CH, CH)],
                                  erows.at[b], sems.at[2 + b]).wait()
            pltpu.make_async_copy(h_hbm.at[idx_s.at[b]], hrows.at[b],
                                  sems.at[4 + b]).wait()

            def row(r, c2):
                for j in range(H // 16):
                    sl = pl.ds(j * 16, 16)
                    erows[b, r, sl] = jnp.maximum(
                        erows[b, r, sl] + hrows[b, r, sl], 0.0)
                return c2
            lax.fori_loop(0, CH, row, 0)
            pltpu.sync_copy(erows.at[b], aggr_sh.at[idx_d.at[b]], add=True)

            @pl.when(k + 2 < NCH)
            def _():
                issue_idx(k + 2, b)
        return carry
    lax.fori_loop(0, NCH // 2, pair, 0)


def _edge_pass_body(src_hbm, dst_hbm, e_a, e_b, h_a, h_b, zr_hbm,
                    out_a, out_b, idx_s, idx_d, erows, hrows, aggr_sh, sems):
    cid = lax.axis_index("c")
    sid = lax.axis_index("s")

    pltpu.sync_copy(zr_hbm, aggr_sh.at[pl.ds(sid * NP, NP)])
    plsc.subcore_barrier()

    def run(e_hbm, h_hbm, out_hbm):
        _edge_chunks(128, C, NCHUNK, sid * EPT, src_hbm, dst_hbm, e_hbm,
                     h_hbm, idx_s, idx_d, erows, hrows, sems, aggr_sh)
        plsc.subcore_barrier()
        pltpu.sync_copy(aggr_sh.at[pl.ds(sid * NP, NP)],
                        out_hbm.at[pl.ds(sid * NP, NP)])

    pl.when(cid == 0)(lambda: run(e_a, h_a, out_a))
    pl.when(cid == 1)(lambda: run(e_b, h_b, out_b))


_edge_pass_128 = pl.kernel(
    _edge_pass_body,
    out_type=(jax.ShapeDtypeStruct((NPAD, 128), jnp.float32),
              jax.ShapeDtypeStruct((NPAD, 128), jnp.float32)),
    mesh=_mesh,
    scratch_types=[
        pltpu.VMEM((2, C), jnp.int32),
        pltpu.VMEM((2, C), jnp.int32),
        pltpu.VMEM((2, C, 128), jnp.float32),
        pltpu.VMEM((2, C, 128), jnp.float32),
        pltpu.VMEM_SHARED((NPAD, 128), jnp.float32),
        pltpu.SemaphoreType.DMA((6,)),
    ],
)


EPT0 = E // (NC * NS)
C0 = 40
NCHUNK0 = EPT0 // C0


def _edge_pass0_body(src_hbm, dst_hbm, e_hbm, h_hbm, zr_hbm,
                     out_p0, out_p1, idx_s, idx_d, erows, hrows,
                     aggr_sh, sems):
    cid = lax.axis_index("c")
    sid = lax.axis_index("s")
    wid = cid * NS + sid

    pltpu.sync_copy(zr_hbm, aggr_sh.at[pl.ds(sid * NP, NP)])
    plsc.subcore_barrier()

    _edge_chunks(128, C0, NCHUNK0, wid * EPT0, src_hbm, dst_hbm, e_hbm,
                 h_hbm, idx_s, idx_d, erows, hrows, sems, aggr_sh)
    plsc.subcore_barrier()

    def out(out_hbm):
        pltpu.sync_copy(aggr_sh.at[pl.ds(sid * NP, NP)],
                        out_hbm.at[pl.ds(sid * NP, NP)])
    pl.when(cid == 0)(lambda: out(out_p0))
    pl.when(cid == 1)(lambda: out(out_p1))


_edge_pass_l0 = pl.kernel(
    _edge_pass0_body,
    out_type=(jax.ShapeDtypeStruct((NPAD, 128), jnp.float32),
              jax.ShapeDtypeStruct((NPAD, 128), jnp.float32)),
    mesh=_mesh,
    scratch_types=[
        pltpu.VMEM((2, C0), jnp.int32),
        pltpu.VMEM((2, C0), jnp.int32),
        pltpu.VMEM((2, C0, 128), jnp.float32),
        pltpu.VMEM((2, C0, 128), jnp.float32),
        pltpu.VMEM_SHARED((NPAD, 128), jnp.float32),
        pltpu.SemaphoreType.DMA((6,)),
    ],
)


CP = 128
RPT = NPAD // NS
GPT = G // NS


def _pool_body(s_a, s_b, batch_hbm, mask_hbm, zr_hbm, zrc_hbm,
               pool_out_a, pool_out_b, cnt_out,
               idxb, rows, ones_rows, pool_sh, cnt_sh, sem):
    cid = lax.axis_index("c")
    sid = lax.axis_index("s")

    pltpu.sync_copy(zr_hbm, pool_sh.at[pl.ds(sid * GPT, GPT)])
    pl.when(cid == 0)(lambda: pltpu.sync_copy(
        zrc_hbm, cnt_sh.at[pl.ds(sid * GPT, GPT)]))
    plsc.subcore_barrier()

    def run(s_hbm):
        def chunk(k, carry):
            base = sid * RPT + k * CP
            pltpu.sync_copy(batch_hbm.at[pl.ds(base, CP)], idxb)
            pltpu.sync_copy(s_hbm.at[pl.ds(base, CP)], rows)
            pltpu.sync_copy(rows, pool_sh.at[idxb], add=True)
            return carry
        lax.fori_loop(0, RPT // CP, chunk, 0)

    def run_counts():
        def chunk(k, carry):
            base = sid * RPT + k * CP
            pltpu.sync_copy(batch_hbm.at[pl.ds(base, CP)], idxb)
            pltpu.sync_copy(mask_hbm.at[pl.ds(base, CP)], ones_rows)
            pltpu.sync_copy(ones_rows, cnt_sh.at[idxb], add=True)
            return carry
        lax.fori_loop(0, RPT // CP, chunk, 0)

    pl.when(cid == 0)(lambda: run(s_a))
    pl.when(cid == 1)(lambda: run(s_b))
    pl.when(cid == 0)(run_counts)
    plsc.subcore_barrier()

    def out(pool_out):
        pltpu.sync_copy(pool_sh.at[pl.ds(sid * GPT, GPT)],
                        pool_out.at[pl.ds(sid * GPT, GPT)])
    pl.when(cid == 0)(lambda: out(pool_out_a))
    pl.when(cid == 1)(lambda: out(pool_out_b))
    pl.when(cid == 0)(lambda: pltpu.sync_copy(
        cnt_sh.at[pl.ds(sid * GPT, GPT)], cnt_out.at[pl.ds(sid * GPT, GPT)]))


_pool = pl.kernel(
    _pool_body,
    out_type=(jax.ShapeDtypeStruct((G, 128), jnp.float32),
              jax.ShapeDtypeStruct((G, 128), jnp.float32),
              jax.ShapeDtypeStruct((G, 128), jnp.float32)),
    mesh=_mesh,
    scratch_types=[
        pltpu.VMEM((CP,), jnp.int32),
        pltpu.VMEM((CP, 128), jnp.float32),
        pltpu.VMEM((CP, 128), jnp.float32),
        pltpu.VMEM_SHARED((G, 128), jnp.float32),
        pltpu.VMEM_SHARED((G, 128), jnp.float32),
        pltpu.SemaphoreType.DMA,
    ],
)


EB = 3200


def _edge_mm_body(attr_ref, we0_ref, we1_ref, we2_ref,
                  e0, e1a, e1b, e2a, e2b):
    attr = attr_ref[...]
    dn = (((1,), (1,)), ((), ()))
    m0 = lax.dot_general(attr, we0_ref[...], dn,
                         preferred_element_type=jnp.float32)
    m1 = lax.dot_general(attr, we1_ref[...], dn,
                         preferred_element_type=jnp.float32)
    m2 = lax.dot_general(attr, we2_ref[...], dn,
                         preferred_element_type=jnp.float32)
    e0[...] = m0
    e1a[...] = m1[:, :128]
    e1b[...] = m1[:, 128:]
    e2a[...] = m2[:, :128]
    e2b[...] = m2[:, 128:]


def _edge_mm(attr, We0, We1, We2):
    grid = (E // EB,)
    return pl.pallas_call(
        _edge_mm_body,
        grid=grid,
        in_specs=[
            pl.BlockSpec((EB, 16), lambda i: (i, 0)),
            pl.BlockSpec((128, 16), lambda i: (0, 0)),
            pl.BlockSpec((256, 16), lambda i: (0, 0)),
            pl.BlockSpec((256, 16), lambda i: (0, 0)),
        ],
        out_specs=[
            pl.BlockSpec((EB, 128), lambda i: (i, 0)),
            pl.BlockSpec((EB, 128), lambda i: (i, 0)),
            pl.BlockSpec((EB, 128), lambda i: (i, 0)),
            pl.BlockSpec((EB, 128), lambda i: (i, 0)),
            pl.BlockSpec((EB, 128), lambda i: (i, 0)),
        ],
        out_shape=[
            jax.ShapeDtypeStruct((E, 128), jnp.float32),
            jax.ShapeDtypeStruct((E, 128), jnp.float32),
            jax.ShapeDtypeStruct((E, 128), jnp.float32),
            jax.ShapeDtypeStruct((E, 128), jnp.float32),
            jax.ShapeDtypeStruct((E, 128), jnp.float32),
        ],
    )(attr, We0, We1, We2)


NB = 2000


def _node_body(first, *refs):
    if first:
        h_a, h_b, g_a, g_b, wa, wb, b2, hn_a, hn_b, sn_a, sn_b = refs
    else:
        (h_a, h_b, g_a, g_b, wa, wb, b2, sp_a, sp_b,
         hn_a, hn_b, sn_a, sn_b) = refs
    dn = (((1,), (1,)), ((), ()))
    t = lax.dot_general(h_a[...] + g_a[...], wa[...], dn,
                        preferred_element_type=jnp.float32)
    t += lax.dot_general(h_b[...] + g_b[...], wb[...], dn,
                         preferred_element_type=jnp.float32)
    r = jnp.maximum(t + b2[...], 0.0)
    ra, rb = r[:, :128], r[:, 128:]
    hn_a[...] = ra
    hn_b[...] = rb
    if first:
        sn_a[...] = ra
        sn_b[...] = rb
    else:
        sn_a[...] = sp_a[...] + ra
        sn_b[...] = sp_b[...] + rb


def _node_update(h_a, h_b, g_a, g_b, Wa, Wb, b2, sp_a=None, sp_b=None,
                 first=False):
    H = h_a.shape[1]
    grid = (N // NB,)
    blk = lambda w: pl.BlockSpec((NB, w), lambda i: (i, 0))
    full = lambda a: pl.BlockSpec(a.shape, lambda i: (0,) * a.ndim)
    args = [h_a, h_b, g_a, g_b, Wa, Wb, b2]
    specs = [blk(H), blk(H), blk(H), blk(H), full(Wa), full(Wb), full(b2)]
    if not first:
        args += [sp_a, sp_b]
        specs += [blk(128), blk(128)]
    return pl.pallas_call(
        functools.partial(_node_body, first),
        grid=grid,
        in_specs=specs,
        out_specs=[blk(128), blk(128), blk(128), blk(128)],
        out_shape=[jax.ShapeDtypeStruct((N, 128), jnp.float32)] * 4,
    )(*args)


def _node0_body(x_ref, p0_ref, p1_ref, wa, wb, b2, hn_a, hn_b, sn_a, sn_b):
    h = x_ref[...] + p0_ref[...] + p1_ref[...]
    dn = (((1,), (1,)), ((), ()))
    t = lax.dot_general(h[:, :64], wa[...], dn,
                        preferred_element_type=jnp.float32)
    t += lax.dot_general(h[:, 64:], wb[...], dn,
                         preferred_element_type=jnp.float32)
    r = jnp.maximum(t + b2[...], 0.0)
    hn_a[...] = r[:, :128]
    hn_b[...] = r[:, 128:]
    sn_a[...] = r[:, :128]
    sn_b[...] = r[:, 128:]


def _node_update0(x, p0, p1, Wa, Wb, b2):
    grid = (N // NB,)
    blk = pl.BlockSpec((NB, 128), lambda i: (i, 0))
    full = lambda a: pl.BlockSpec(a.shape, lambda i: (0,) * a.ndim)
    return pl.pallas_call(
        _node0_body,
        grid=grid,
        in_specs=[blk, blk, blk, full(Wa), full(Wb), full(b2)],
        out_specs=[blk, blk, blk, blk],
        out_shape=[jax.ShapeDtypeStruct((N, 128), jnp.float32)] * 4,
    )(x, p0, p1, Wa, Wb, b2)


def _scale_body(pa_ref, pb_ref, cnt_ref, out_ref):
    cnt = cnt_ref[:, 0:1]
    scale = jnp.where(cnt > 0.0, lax.rsqrt(jnp.maximum(cnt, 1.0)), 0.0)
    out_ref[:, :128] = pa_ref[...] * scale
    out_ref[:, 128:] = pb_ref[...] * scale


def _scale(pool_a, pool_b, cnt):
    return pl.pallas_call(
        _scale_body,
        out_shape=jax.ShapeDtypeStruct((G, 256), jnp.float32),
    )(pool_a, pool_b, cnt)


def kernel(x, edge_index, edge_attr, batch, W0, b0, We0, W1, b1, We1,
           W2, b2, We2):
    src = edge_index[0]
    dst = edge_index[1]

    zr128 = jnp.zeros((NP, 128), jnp.float32)
    zr_g = jnp.zeros((GPT, 128), jnp.float32)
    zr_gc = jnp.zeros((GPT, 128), jnp.float32)
    batch_pad = jnp.concatenate(
        [batch, jnp.full((NPAD - N,), G - 1, jnp.int32)])
    mask_pad = jnp.concatenate(
        [jnp.ones((N, 128), jnp.float32), jnp.zeros((NPAD - N, 128),
                                                    jnp.float32)])

    e0, e1a, e1b, e2a, e2b = _edge_mm(edge_attr, We0, We1, We2)

    p0, p1 = _edge_pass_l0(src, dst, e0, x, zr128)
    h1a, h1b, s1a, s1b = _node_update0(
        x, p0, p1, W0[:, :64], W0[:, 64:], b0[None, :])

    g1a, g1b = _edge_pass_128(src, dst, e1a, e1b, h1a, h1b, zr128)
    h2a, h2b, s2a, s2b = _node_update(
        h1a, h1b, g1a, g1b, W1[:, :128], W1[:, 128:], b1[None, :],
        s1a, s1b, first=False)

    g2a, g2b = _edge_pass_128(src, dst, e2a, e2b, h2a, h2b, zr128)
    _, _, s3a, s3b = _node_update(
        h2a, h2b, g2a, g2b, W2[:, :128], W2[:, 128:], b2[None, :],
        s2a, s2b, first=False)

    s3a_pad = jnp.concatenate([s3a, jnp.zeros((NPAD - N, 128), jnp.float32)])
    s3b_pad = jnp.concatenate([s3b, jnp.zeros((NPAD - N, 128), jnp.float32)])

    pool_a, pool_b, cnt = _pool(s3a_pad, s3b_pad, batch_pad, mask_pad,
                                zr_g, zr_gc)
    return _scale(pool_a, pool_b, cnt)


# speedup vs baseline: 3.7229x; 1.0014x over previous
import functools

import jax
import jax.numpy as jnp
from jax import lax
from jax.experimental import pallas as pl
from jax.experimental.pallas import tpu as pltpu
from jax.experimental.pallas import tpu_sc as plsc

N = 10000
E = 320000
G = 128
NPAD = 10240

NC = 2
NS = 16
NP = NPAD // NS
EPT = E // NS
C = 80
NCHUNK = EPT // C

_mesh = plsc.VectorSubcoreMesh(core_axis_name="c", subcore_axis_name="s")


def _edge_chunks(H, CH, NCH, base, src_hbm, dst_hbm, e_hbm, h_hbm,
                 idx_s, idx_d, erows, hrows, sems, aggr_sh):
    def issue_idx(k, b):
        pltpu.async_copy(src_hbm.at[pl.ds(base + k * CH, CH)],
                         idx_s.at[b], sems.at[b])
        pltpu.async_copy(dst_hbm.at[pl.ds(base + k * CH, CH)],
                         idx_d.at[b], sems.at[b])
        pltpu.async_copy(e_hbm.at[pl.ds(base + k * CH, CH)],
                         erows.at[b], sems.at[2 + b])

    def wait_idx(k, b):
        pltpu.make_async_copy(src_hbm.at[pl.ds(base + k * CH, CH)],
                              idx_s.at[b], sems.at[b]).wait()
        pltpu.make_async_copy(dst_hbm.at[pl.ds(base + k * CH, CH)],
                              idx_d.at[b], sems.at[b]).wait()

    def issue_gather(b):
        pltpu.async_copy(h_hbm.at[idx_s.at[b]], hrows.at[b], sems.at[4 + b])

    issue_idx(0, 0)
    issue_idx(1, 1)
    wait_idx(0, 0)
    issue_gather(0)

    def pair(kk, carry):
        for b in range(2):
            k = kk * 2 + b
            b1 = 1 - b

            @pl.when(k + 1 < NCH)
            def _():
                wait_idx(k + 1, b1)
                issue_gather(b1)

            pltpu.make_async_copy(e_hbm.at[pl.ds(base + k * CH, CH)],
                                  erows.at[b], sems.at[2 + b]).wait()
            pltpu.make_async_copy(h_hbm.at[idx_s.at[b]], hrows.at[b],
                                  sems.at[4 + b]).wait()

            def row(r4, c2):
                for u in range(4):
                    r = r4 * 4 + u
                    for j in range(H // 16):
                        sl = pl.ds(j * 16, 16)
                        erows[b, r, sl] = jnp.maximum(
                            erows[b, r, sl] + hrows[b, r, sl], 0.0)
                return c2
            lax.fori_loop(0, CH // 4, row, 0)
            pltpu.sync_copy(erows.at[b], aggr_sh.at[idx_d.at[b]], add=True)

            @pl.when(k + 2 < NCH)
            def _():
                issue_idx(k + 2, b)
        return carry
    lax.fori_loop(0, NCH // 2, pair, 0)


def _edge_pass_body(src_hbm, dst_hbm, e_a, e_b, h_a, h_b, zr_hbm,
                    out_a, out_b, idx_s, idx_d, erows, hrows, aggr_sh, sems):
    cid = lax.axis_index("c")
    sid = lax.axis_index("s")

    pltpu.sync_copy(zr_hbm, aggr_sh.at[pl.ds(sid * NP, NP)])
    plsc.subcore_barrier()

    def run(e_hbm, h_hbm, out_hbm):
        _edge_chunks(128, C, NCHUNK, sid * EPT, src_hbm, dst_hbm, e_hbm,
                     h_hbm, idx_s, idx_d, erows, hrows, sems, aggr_sh)
        plsc.subcore_barrier()
        pltpu.sync_copy(aggr_sh.at[pl.ds(sid * NP, NP)],
                        out_hbm.at[pl.ds(sid * NP, NP)])

    pl.when(cid == 0)(lambda: run(e_a, h_a, out_a))
    pl.when(cid == 1)(lambda: run(e_b, h_b, out_b))


_edge_pass_128 = pl.kernel(
    _edge_pass_body,
    out_type=(jax.ShapeDtypeStruct((NPAD, 128), jnp.float32),
              jax.ShapeDtypeStruct((NPAD, 128), jnp.float32)),
    mesh=_mesh,
    scratch_types=[
        pltpu.VMEM((2, C), jnp.int32),
        pltpu.VMEM((2, C), jnp.int32),
        pltpu.VMEM((2, C, 128), jnp.float32),
        pltpu.VMEM((2, C, 128), jnp.float32),
        pltpu.VMEM_SHARED((NPAD, 128), jnp.float32),
        pltpu.SemaphoreType.DMA((6,)),
    ],
)


EPT0 = E // (NC * NS)
C0 = 40
NCHUNK0 = EPT0 // C0


def _edge_pass0_body(src_hbm, dst_hbm, e_hbm, h_hbm, zr_hbm,
                     out_p0, out_p1, idx_s, idx_d, erows, hrows,
                     aggr_sh, sems):
    cid = lax.axis_index("c")
    sid = lax.axis_index("s")
    wid = cid * NS + sid

    pltpu.sync_copy(zr_hbm, aggr_sh.at[pl.ds(sid * NP, NP)])
    plsc.subcore_barrier()

    _edge_chunks(128, C0, NCHUNK0, wid * EPT0, src_hbm, dst_hbm, e_hbm,
                 h_hbm, idx_s, idx_d, erows, hrows, sems, aggr_sh)
    plsc.subcore_barrier()

    def out(out_hbm):
        pltpu.sync_copy(aggr_sh.at[pl.ds(sid * NP, NP)],
                        out_hbm.at[pl.ds(sid * NP, NP)])
    pl.when(cid == 0)(lambda: out(out_p0))
    pl.when(cid == 1)(lambda: out(out_p1))


_edge_pass_l0 = pl.kernel(
    _edge_pass0_body,
    out_type=(jax.ShapeDtypeStruct((NPAD, 128), jnp.float32),
              jax.ShapeDtypeStruct((NPAD, 128), jnp.float32)),
    mesh=_mesh,
    scratch_types=[
        pltpu.VMEM((2, C0), jnp.int32),
        pltpu.VMEM((2, C0), jnp.int32),
        pltpu.VMEM((2, C0, 128), jnp.float32),
        pltpu.VMEM((2, C0, 128), jnp.float32),
        pltpu.VMEM_SHARED((NPAD, 128), jnp.float32),
        pltpu.SemaphoreType.DMA((6,)),
    ],
)


CP = 128
RPT = NPAD // NS
GPT = G // NS


def _pool_body(s_a, s_b, batch_hbm, mask_hbm, zr_hbm, zrc_hbm,
               pool_out_a, pool_out_b, cnt_out,
               idxb, rows, ones_rows, pool_sh, cnt_sh, sem):
    cid = lax.axis_index("c")
    sid = lax.axis_index("s")

    pltpu.sync_copy(zr_hbm, pool_sh.at[pl.ds(sid * GPT, GPT)])
    pl.when(cid == 0)(lambda: pltpu.sync_copy(
        zrc_hbm, cnt_sh.at[pl.ds(sid * GPT, GPT)]))
    plsc.subcore_barrier()

    def run(s_hbm):
        def chunk(k, carry):
            base = sid * RPT + k * CP
            pltpu.sync_copy(batch_hbm.at[pl.ds(base, CP)], idxb)
            pltpu.sync_copy(s_hbm.at[pl.ds(base, CP)], rows)
            pltpu.sync_copy(rows, pool_sh.at[idxb], add=True)
            return carry
        lax.fori_loop(0, RPT // CP, chunk, 0)

    def run_counts():
        def chunk(k, carry):
            base = sid * RPT + k * CP
            pltpu.sync_copy(batch_hbm.at[pl.ds(base, CP)], idxb)
            pltpu.sync_copy(mask_hbm.at[pl.ds(base, CP)], ones_rows)
            pltpu.sync_copy(ones_rows, cnt_sh.at[idxb], add=True)
            return carry
        lax.fori_loop(0, RPT // CP, chunk, 0)

    pl.when(cid == 0)(lambda: run(s_a))
    pl.when(cid == 1)(lambda: run(s_b))
    pl.when(cid == 0)(run_counts)
    plsc.subcore_barrier()

    def out(pool_out):
        pltpu.sync_copy(pool_sh.at[pl.ds(sid * GPT, GPT)],
                        pool_out.at[pl.ds(sid * GPT, GPT)])
    pl.when(cid == 0)(lambda: out(pool_out_a))
    pl.when(cid == 1)(lambda: out(pool_out_b))
    pl.when(cid == 0)(lambda: pltpu.sync_copy(
        cnt_sh.at[pl.ds(sid * GPT, GPT)], cnt_out.at[pl.ds(sid * GPT, GPT)]))


_pool = pl.kernel(
    _pool_body,
    out_type=(jax.ShapeDtypeStruct((G, 128), jnp.float32),
              jax.ShapeDtypeStruct((G, 128), jnp.float32),
              jax.ShapeDtypeStruct((G, 128), jnp.float32)),
    mesh=_mesh,
    scratch_types=[
        pltpu.VMEM((CP,), jnp.int32),
        pltpu.VMEM((CP, 128), jnp.float32),
        pltpu.VMEM((CP, 128), jnp.float32),
        pltpu.VMEM_SHARED((G, 128), jnp.float32),
        pltpu.VMEM_SHARED((G, 128), jnp.float32),
        pltpu.SemaphoreType.DMA,
    ],
)


EB = 3200


def _edge_mm0_body(attr_ref, we_ref, e0):
    dn = (((1,), (1,)), ((), ()))
    e0[...] = lax.dot_general(attr_ref[...], we_ref[...], dn,
                              preferred_element_type=jnp.float32)


def _edge_mm12_body(attr_ref, we_ref, ea, eb):
    dn = (((1,), (1,)), ((), ()))
    m = lax.dot_general(attr_ref[...], we_ref[...], dn,
                        preferred_element_type=jnp.float32)
    ea[...] = m[:, :128]
    eb[...] = m[:, 128:]


def _edge_mm0(attr, We0):
    return pl.pallas_call(
        _edge_mm0_body,
        grid=(E // EB,),
        in_specs=[pl.BlockSpec((EB, 16), lambda i: (i, 0)),
                  pl.BlockSpec((128, 16), lambda i: (0, 0))],
        out_specs=pl.BlockSpec((EB, 128), lambda i: (i, 0)),
        out_shape=jax.ShapeDtypeStruct((E, 128), jnp.float32),
    )(attr, We0)


def _edge_mm12(attr, We):
    return pl.pallas_call(
        _edge_mm12_body,
        grid=(E // EB,),
        in_specs=[pl.BlockSpec((EB, 16), lambda i: (i, 0)),
                  pl.BlockSpec((256, 16), lambda i: (0, 0))],
        out_specs=[pl.BlockSpec((EB, 128), lambda i: (i, 0)),
                   pl.BlockSpec((EB, 128), lambda i: (i, 0))],
        out_shape=[jax.ShapeDtypeStruct((E, 128), jnp.float32)] * 2,
    )(attr, We)


NB = 2000


def _node_body(first, *refs):
    if first:
        h_a, h_b, g_a, g_b, wa, wb, b2, hn_a, hn_b, sn_a, sn_b = refs
    else:
        (h_a, h_b, g_a, g_b, wa, wb, b2, sp_a, sp_b,
         hn_a, hn_b, sn_a, sn_b) = refs
    dn = (((1,), (1,)), ((), ()))
    t = lax.dot_general(h_a[...] + g_a[...], wa[...], dn,
                        preferred_element_type=jnp.float32)
    t += lax.dot_general(h_b[...] + g_b[...], wb[...], dn,
                         preferred_element_type=jnp.float32)
    r = jnp.maximum(t + b2[...], 0.0)
    ra, rb = r[:, :128], r[:, 128:]
    hn_a[...] = ra
    hn_b[...] = rb
    if first:
        sn_a[...] = ra
        sn_b[...] = rb
    else:
        sn_a[...] = sp_a[...] + ra
        sn_b[...] = sp_b[...] + rb


def _node_update(h_a, h_b, g_a, g_b, Wa, Wb, b2, sp_a=None, sp_b=None,
                 first=False):
    H = h_a.shape[1]
    grid = (N // NB,)
    blk = lambda w: pl.BlockSpec((NB, w), lambda i: (i, 0))
    full = lambda a: pl.BlockSpec(a.shape, lambda i: (0,) * a.ndim)
    args = [h_a, h_b, g_a, g_b, Wa, Wb, b2]
    specs = [blk(H), blk(H), blk(H), blk(H), full(Wa), full(Wb), full(b2)]
    if not first:
        args += [sp_a, sp_b]
        specs += [blk(128), blk(128)]
    return pl.pallas_call(
        functools.partial(_node_body, first),
        grid=grid,
        in_specs=specs,
        out_specs=[blk(128), blk(128), blk(128), blk(128)],
        out_shape=[jax.ShapeDtypeStruct((N, 128), jnp.float32)] * 4,
    )(*args)


def _node0_body(x_ref, p0_ref, p1_ref, wa, wb, b2, hn_a, hn_b, sn_a, sn_b):
    h = x_ref[...] + p0_ref[...] + p1_ref[...]
    dn = (((1,), (1,)), ((), ()))
    t = lax.dot_general(h[:, :64], wa[...], dn,
                        preferred_element_type=jnp.float32)
    t += lax.dot_general(h[:, 64:], wb[...], dn,
                         preferred_element_type=jnp.float32)
    r = jnp.maximum(t + b2[...], 0.0)
    hn_a[...] = r[:, :128]
    hn_b[...] = r[:, 128:]
    sn_a[...] = r[:, :128]
    sn_b[...] = r[:, 128:]


def _node_update0(x, p0, p1, Wa, Wb, b2):
    grid = (N // NB,)
    blk = pl.BlockSpec((NB, 128), lambda i: (i, 0))
    full = lambda a: pl.BlockSpec(a.shape, lambda i: (0,) * a.ndim)
    return pl.pallas_call(
        _node0_body,
        grid=grid,
        in_specs=[blk, blk, blk, full(Wa), full(Wb), full(b2)],
        out_specs=[blk, blk, blk, blk],
        out_shape=[jax.ShapeDtypeStruct((N, 128), jnp.float32)] * 4,
    )(x, p0, p1, Wa, Wb, b2)


def _scale_body(pa_ref, pb_ref, cnt_ref, out_ref):
    cnt = cnt_ref[:, 0:1]
    scale = jnp.where(cnt > 0.0, lax.rsqrt(jnp.maximum(cnt, 1.0)), 0.0)
    out_ref[:, :128] = pa_ref[...] * scale
    out_ref[:, 128:] = pb_ref[...] * scale


def _scale(pool_a, pool_b, cnt):
    return pl.pallas_call(
        _scale_body,
        out_shape=jax.ShapeDtypeStruct((G, 256), jnp.float32),
    )(pool_a, pool_b, cnt)


def kernel(x, edge_index, edge_attr, batch, W0, b0, We0, W1, b1, We1,
           W2, b2, We2):
    src = edge_index[0]
    dst = edge_index[1]

    zr128 = jnp.zeros((NP, 128), jnp.float32)
    zr_g = jnp.zeros((GPT, 128), jnp.float32)
    zr_gc = jnp.zeros((GPT, 128), jnp.float32)
    batch_pad = jnp.concatenate(
        [batch, jnp.full((NPAD - N,), G - 1, jnp.int32)])
    mask_pad = jnp.concatenate(
        [jnp.ones((N, 128), jnp.float32), jnp.zeros((NPAD - N, 128),
                                                    jnp.float32)])

    e0 = _edge_mm0(edge_attr, We0)
    e1a, e1b = _edge_mm12(edge_attr, We1)
    e2a, e2b = _edge_mm12(edge_attr, We2)

    p0, p1 = _edge_pass_l0(src, dst, e0, x, zr128)
    h1a, h1b, s1a, s1b = _node_update0(
        x, p0, p1, W0[:, :64], W0[:, 64:], b0[None, :])

    g1a, g1b = _edge_pass_128(src, dst, e1a, e1b, h1a, h1b, zr128)
    h2a, h2b, s2a, s2b = _node_update(
        h1a, h1b, g1a, g1b, W1[:, :128], W1[:, 128:], b1[None, :],
        s1a, s1b, first=False)

    g2a, g2b = _edge_pass_128(src, dst, e2a, e2b, h2a, h2b, zr128)
    _, _, s3a, s3b = _node_update(
        h2a, h2b, g2a, g2b, W2[:, :128], W2[:, 128:], b2[None, :],
        s2a, s2b, first=False)

    s3a_pad = jnp.concatenate([s3a, jnp.zeros((NPAD - N, 128), jnp.float32)])
    s3b_pad = jnp.concatenate([s3b, jnp.zeros((NPAD - N, 128), jnp.float32)])

    pool_a, pool_b, cnt = _pool(s3a_pad, s3b_pad, batch_pad, mask_pad,
                                zr_g, zr_gc)
    return _scale(pool_a, pool_b, cnt)


# speedup vs baseline: 3.8345x; 1.0300x over previous
import functools

import jax
import jax.numpy as jnp
from jax import lax
from jax.experimental import pallas as pl
from jax.experimental.pallas import tpu as pltpu
from jax.experimental.pallas import tpu_sc as plsc

N = 10000
E = 320000
G = 128
NPAD = 10240

NC = 2
NS = 16
NP = NPAD // NS
EPT = E // NS
C = 80
NCHUNK = EPT // C

_mesh = plsc.VectorSubcoreMesh(core_axis_name="c", subcore_axis_name="s")


def _edge_chunks(H, CH, NCH, base, src_hbm, dst_hbm, e_hbm, h_hbm,
                 idx_s, idx_d, erows, hrows, sems, aggr_sh):
    def issue_idx(k, b):
        pltpu.async_copy(src_hbm.at[pl.ds(base + k * CH, CH)],
                         idx_s.at[b], sems.at[b])
        pltpu.async_copy(dst_hbm.at[pl.ds(base + k * CH, CH)],
                         idx_d.at[b], sems.at[b])
        pltpu.async_copy(e_hbm.at[pl.ds(base + k * CH, CH)],
                         erows.at[b], sems.at[2 + b])

    def wait_idx(k, b):
        pltpu.make_async_copy(src_hbm.at[pl.ds(base + k * CH, CH)],
                              idx_s.at[b], sems.at[b]).wait()
        pltpu.make_async_copy(dst_hbm.at[pl.ds(base + k * CH, CH)],
                              idx_d.at[b], sems.at[b]).wait()

    def issue_gather(b):
        pltpu.async_copy(h_hbm.at[idx_s.at[b]], hrows.at[b], sems.at[4 + b])

    def chunk_body(k, b):
        pltpu.make_async_copy(e_hbm.at[pl.ds(base + k * CH, CH)],
                              erows.at[b], sems.at[2 + b]).wait()
        pltpu.make_async_copy(h_hbm.at[idx_s.at[b]], hrows.at[b],
                              sems.at[4 + b]).wait()

        def row(r4, c2):
            for u in range(2):
                r = r4 * 2 + u
                for j in range(H // 16):
                    sl = pl.ds(j * 16, 16)
                    erows[b, r, sl] = jnp.maximum(
                        erows[b, r, sl] + hrows[b, r, sl], 0.0)
            return c2
        lax.fori_loop(0, CH // 2, row, 0)
        pltpu.sync_copy(erows.at[b], aggr_sh.at[idx_d.at[b]], add=True)

    issue_idx(0, 0)
    issue_idx(1, 1)
    wait_idx(0, 0)
    issue_gather(0)

    def pair(kk, carry):
        for b in range(2):
            k = kk * 2 + b
            b1 = 1 - b

            @pl.when(k + 1 < NCH)
            def _():
                wait_idx(k + 1, b1)
                issue_gather(b1)

            chunk_body(k, b)

            @pl.when(k + 2 < NCH)
            def _():
                issue_idx(k + 2, b)
        return carry
    lax.fori_loop(0, NCH // 2, pair, 0)
    if NCH % 2:
        chunk_body(NCH - 1, 0)


def _edge_pass_body(src_hbm, dst_hbm, e_a, e_b, h_a, h_b, zr_hbm,
                    out_a, out_b, idx_s, idx_d, erows, hrows, aggr_sh,
                    sems):
    cid = lax.axis_index("c")
    sid = lax.axis_index("s")

    pltpu.sync_copy(zr_hbm, aggr_sh.at[pl.ds(sid * NP, NP)])
    plsc.subcore_barrier()

    def run(e_hbm, h_hbm, out_hbm):
        _edge_chunks(128, C, NCHUNK, sid * EPT, src_hbm, dst_hbm, e_hbm,
                     h_hbm, idx_s, idx_d, erows, hrows, sems, aggr_sh)
        plsc.subcore_barrier()
        pltpu.sync_copy(aggr_sh.at[pl.ds(sid * NP, NP)],
                        out_hbm.at[pl.ds(sid * NP, NP)])

    pl.when(cid == 0)(lambda: run(e_a, h_a, out_a))
    pl.when(cid == 1)(lambda: run(e_b, h_b, out_b))


_edge_pass_128 = pl.kernel(
    _edge_pass_body,
    out_type=(jax.ShapeDtypeStruct((NPAD, 128), jnp.float32),
              jax.ShapeDtypeStruct((NPAD, 128), jnp.float32)),
    mesh=_mesh,
    scratch_types=[
        pltpu.VMEM((2, C), jnp.int32),
        pltpu.VMEM((2, C), jnp.int32),
        pltpu.VMEM((2, C, 128), jnp.float32),
        pltpu.VMEM((2, C, 128), jnp.float32),
        pltpu.VMEM_SHARED((NPAD, 128), jnp.float32),
        pltpu.SemaphoreType.DMA((6,)),
    ],
)


EPT0 = E // (NC * NS)
C0 = 80
NCHUNK0 = EPT0 // C0


def _edge_pass0_body(src_hbm, dst_hbm, e_hbm, h_hbm, zr_hbm,
                     out_p0, out_p1, idx_s, idx_d, erows, hrows,
                     aggr_sh, sems):
    cid = lax.axis_index("c")
    sid = lax.axis_index("s")
    wid = cid * NS + sid

    pltpu.sync_copy(zr_hbm, aggr_sh.at[pl.ds(sid * NP, NP)])
    plsc.subcore_barrier()

    _edge_chunks(128, C0, NCHUNK0, wid * EPT0, src_hbm, dst_hbm, e_hbm,
                 h_hbm, idx_s, idx_d, erows, hrows, sems, aggr_sh)
    plsc.subcore_barrier()

    def out(out_hbm):
        pltpu.sync_copy(aggr_sh.at[pl.ds(sid * NP, NP)],
                        out_hbm.at[pl.ds(sid * NP, NP)])
    pl.when(cid == 0)(lambda: out(out_p0))
    pl.when(cid == 1)(lambda: out(out_p1))


_edge_pass_l0 = pl.kernel(
    _edge_pass0_body,
    out_type=(jax.ShapeDtypeStruct((NPAD, 128), jnp.float32),
              jax.ShapeDtypeStruct((NPAD, 128), jnp.float32)),
    mesh=_mesh,
    scratch_types=[
        pltpu.VMEM((2, C0), jnp.int32),
        pltpu.VMEM((2, C0), jnp.int32),
        pltpu.VMEM((2, C0, 128), jnp.float32),
        pltpu.VMEM((2, C0, 128), jnp.float32),
        pltpu.VMEM_SHARED((NPAD, 128), jnp.float32),
        pltpu.SemaphoreType.DMA((6,)),
    ],
)


CP = 128
RPT = NPAD // NS
GPT = G // NS


def _pool_body(s_a, s_b, batch_hbm, mask_hbm, zr_hbm, zrc_hbm,
               pool_out_a, pool_out_b, cnt_out,
               idxb, rows, ones_rows, pool_sh, cnt_sh, sem):
    cid = lax.axis_index("c")
    sid = lax.axis_index("s")

    pltpu.sync_copy(zr_hbm, pool_sh.at[pl.ds(sid * GPT, GPT)])
    pl.when(cid == 0)(lambda: pltpu.sync_copy(
        zrc_hbm, cnt_sh.at[pl.ds(sid * GPT, GPT)]))
    plsc.subcore_barrier()

    def run(s_hbm):
        def chunk(k, carry):
            base = sid * RPT + k * CP
            pltpu.sync_copy(batch_hbm.at[pl.ds(base, CP)], idxb)
            pltpu.sync_copy(s_hbm.at[pl.ds(base, CP)], rows)
            pltpu.sync_copy(rows, pool_sh.at[idxb], add=True)
            return carry
        lax.fori_loop(0, RPT // CP, chunk, 0)

    def run_counts():
        def chunk(k, carry):
            base = sid * RPT + k * CP
            pltpu.sync_copy(batch_hbm.at[pl.ds(base, CP)], idxb)
            pltpu.sync_copy(mask_hbm.at[pl.ds(base, CP)], ones_rows)
            pltpu.sync_copy(ones_rows, cnt_sh.at[idxb], add=True)
            return carry
        lax.fori_loop(0, RPT // CP, chunk, 0)

    pl.when(cid == 0)(lambda: run(s_a))
    pl.when(cid == 1)(lambda: run(s_b))
    pl.when(cid == 0)(run_counts)
    plsc.subcore_barrier()

    def out(pool_out):
        pltpu.sync_copy(pool_sh.at[pl.ds(sid * GPT, GPT)],
                        pool_out.at[pl.ds(sid * GPT, GPT)])
    pl.when(cid == 0)(lambda: out(pool_out_a))
    pl.when(cid == 1)(lambda: out(pool_out_b))
    pl.when(cid == 0)(lambda: pltpu.sync_copy(
        cnt_sh.at[pl.ds(sid * GPT, GPT)], cnt_out.at[pl.ds(sid * GPT, GPT)]))


_pool = pl.kernel(
    _pool_body,
    out_type=(jax.ShapeDtypeStruct((G, 128), jnp.float32),
              jax.ShapeDtypeStruct((G, 128), jnp.float32),
              jax.ShapeDtypeStruct((G, 128), jnp.float32)),
    mesh=_mesh,
    scratch_types=[
        pltpu.VMEM((CP,), jnp.int32),
        pltpu.VMEM((CP, 128), jnp.float32),
        pltpu.VMEM((CP, 128), jnp.float32),
        pltpu.VMEM_SHARED((G, 128), jnp.float32),
        pltpu.VMEM_SHARED((G, 128), jnp.float32),
        pltpu.SemaphoreType.DMA,
    ],
)


EB = 3200


def _edge_mm0_body(attr_ref, we_ref, e0):
    dn = (((1,), (1,)), ((), ()))
    e0[...] = lax.dot_general(attr_ref[...], we_ref[...], dn,
                              preferred_element_type=jnp.float32)


def _edge_mm12_body(attr_ref, we_ref, ea, eb):
    dn = (((1,), (1,)), ((), ()))
    m = lax.dot_general(attr_ref[...], we_ref[...], dn,
                        preferred_element_type=jnp.float32)
    ea[...] = m[:, :128]
    eb[...] = m[:, 128:]


def _edge_mm0(attr, We0p):
    return pl.pallas_call(
        _edge_mm0_body,
        grid=(E // EB,),
        in_specs=[pl.BlockSpec((EB, 16), lambda i: (i, 0)),
                  pl.BlockSpec((128, 16), lambda i: (0, 0))],
        out_specs=pl.BlockSpec((EB, 128), lambda i: (i, 0)),
        out_shape=jax.ShapeDtypeStruct((E, 128), jnp.float32),
    )(attr, We0p)


def _edge_mm12(attr, Wep):
    return pl.pallas_call(
        _edge_mm12_body,
        grid=(E // EB,),
        in_specs=[pl.BlockSpec((EB, 16), lambda i: (i, 0)),
                  pl.BlockSpec((256, 16), lambda i: (0, 0))],
        out_specs=[pl.BlockSpec((EB, 128), lambda i: (i, 0)),
                   pl.BlockSpec((EB, 128), lambda i: (i, 0))],
        out_shape=[jax.ShapeDtypeStruct((E, 128), jnp.float32)] * 2,
    )(attr, Wep)


NB = 2000


def _node_body(first, *refs):
    if first:
        h_a, h_b, g_a, g_b, wa, wb, b2, hn_a, hn_b, sn_a, sn_b = refs
    else:
        (h_a, h_b, g_a, g_b, wa, wb, b2, sp_a, sp_b,
         hn_a, hn_b, sn_a, sn_b) = refs
    dn = (((1,), (1,)), ((), ()))
    t = lax.dot_general(h_a[...] + g_a[...], wa[...], dn,
                        preferred_element_type=jnp.float32)
    t += lax.dot_general(h_b[...] + g_b[...], wb[...], dn,
                         preferred_element_type=jnp.float32)
    r = jnp.maximum(t + b2[...], 0.0)
    ra, rb = r[:, :128], r[:, 128:]
    hn_a[...] = ra
    hn_b[...] = rb
    if first:
        sn_a[...] = ra
        sn_b[...] = rb
    else:
        sn_a[...] = sp_a[...] + ra
        sn_b[...] = sp_b[...] + rb


def _node_update(h_a, h_b, g_a, g_b, Wa, Wb, b2, sp_a=None, sp_b=None,
                 first=False):
    H = h_a.shape[1]
    grid = (N // NB,)
    blk = lambda w: pl.BlockSpec((NB, w), lambda i: (i, 0))
    full = lambda a: pl.BlockSpec(a.shape, lambda i: (0,) * a.ndim)
    args = [h_a, h_b, g_a, g_b, Wa, Wb, b2]
    specs = [blk(H), blk(H), blk(H), blk(H), full(Wa), full(Wb), full(b2)]
    if not first:
        args += [sp_a, sp_b]
        specs += [blk(128), blk(128)]
    return pl.pallas_call(
        functools.partial(_node_body, first),
        grid=grid,
        in_specs=specs,
        out_specs=[blk(128), blk(128), blk(128), blk(128)],
        out_shape=[jax.ShapeDtypeStruct((N, 128), jnp.float32)] * 4,
    )(*args)


def _node0_body(x_ref, p0_ref, p1_ref, wa, wb, b2, hn_a, hn_b, sn_a, sn_b):
    h = x_ref[...] + p0_ref[...] + p1_ref[...]
    dn = (((1,), (1,)), ((), ()))
    t = lax.dot_general(h[:, :64], wa[...], dn,
                        preferred_element_type=jnp.float32)
    t += lax.dot_general(h[:, 64:], wb[...], dn,
                         preferred_element_type=jnp.float32)
    r = jnp.maximum(t + b2[...], 0.0)
    hn_a[...] = r[:, :128]
    hn_b[...] = r[:, 128:]
    sn_a[...] = r[:, :128]
    sn_b[...] = r[:, 128:]


def _node_update0(x, p0, p1, Wa, Wb, b2):
    grid = (N // NB,)
    blk = pl.BlockSpec((NB, 128), lambda i: (i, 0))
    full = lambda a: pl.BlockSpec(a.shape, lambda i: (0,) * a.ndim)
    return pl.pallas_call(
        _node0_body,
        grid=grid,
        in_specs=[blk, blk, blk, full(Wa), full(Wb), full(b2)],
        out_specs=[blk, blk, blk, blk],
        out_shape=[jax.ShapeDtypeStruct((N, 128), jnp.float32)] * 4,
    )(x, p0, p1, Wa, Wb, b2)


def _scale_body(pa_ref, pb_ref, cnt_ref, out_ref):
    cnt = cnt_ref[:, 0:1]
    scale = jnp.where(cnt > 0.0, lax.rsqrt(jnp.maximum(cnt, 1.0)), 0.0)
    out_ref[:, :128] = pa_ref[...] * scale
    out_ref[:, 128:] = pb_ref[...] * scale


def _scale(pool_a, pool_b, cnt):
    return pl.pallas_call(
        _scale_body,
        out_shape=jax.ShapeDtypeStruct((G, 256), jnp.float32),
    )(pool_a, pool_b, cnt)


def kernel(x, edge_index, edge_attr, batch, W0, b0, We0, W1, b1, We1,
           W2, b2, We2):
    src = edge_index[0]
    dst = edge_index[1]

    zr128 = jnp.zeros((NP, 128), jnp.float32)
    zr_g = jnp.zeros((GPT, 128), jnp.float32)
    zr_gc = jnp.zeros((GPT, 128), jnp.float32)
    batch_pad = jnp.concatenate(
        [batch, jnp.full((NPAD - N,), G - 1, jnp.int32)])
    mask_pad = jnp.concatenate(
        [jnp.ones((N, 128), jnp.float32), jnp.zeros((NPAD - N, 128),
                                                    jnp.float32)])
    e0 = _edge_mm0(edge_attr, We0)
    e1a, e1b = _edge_mm12(edge_attr, We1)
    e2a, e2b = _edge_mm12(edge_attr, We2)

    p0, p1 = _edge_pass_l0(src, dst, e0, x, zr128)
    h1a, h1b, s1a, s1b = _node_update0(
        x, p0, p1, W0[:, :64], W0[:, 64:], b0[None, :])

    g1a, g1b = _edge_pass_128(src, dst, e1a, e1b, h1a, h1b, zr128)
    h2a, h2b, s2a, s2b = _node_update(
        h1a, h1b, g1a, g1b, W1[:, :128], W1[:, 128:], b1[None, :],
        s1a, s1b, first=False)

    g2a, g2b = _edge_pass_128(src, dst, e2a, e2b, h2a, h2b, zr128)
    _, _, s3a, s3b = _node_update(
        h2a, h2b, g2a, g2b, W2[:, :128], W2[:, 128:], b2[None, :],
        s2a, s2b, first=False)

    s3a_pad = jnp.concatenate([s3a, jnp.zeros((NPAD - N, 128), jnp.float32)])
    s3b_pad = jnp.concatenate([s3b, jnp.zeros((NPAD - N, 128), jnp.float32)])

    pool_a, pool_b, cnt = _pool(s3a_pad, s3b_pad, batch_pad, mask_pad,
                                zr_g, zr_gc)
    return _scale(pool_a, pool_b, cnt)
